# baseline (speedup 1.0000x reference)
import functools
import jax
import jax.numpy as jnp
from jax.experimental import pallas as pl
from jax.experimental.pallas import tpu as pltpu

HID = 128
LAT = 16
NRBF = 16
CUT = 10.0
NL = 3
NN = 10000
ME = 1 << 20
NP = 10240
NBK = 256
EBK = 1024


def _embed_body(t_ref, ff_ref, Wi_ref, bi_ref, Wf_ref, bf_ref, o_ref):
    t = t_ref[...]
    oh = (t == jax.lax.broadcasted_iota(jnp.int32, (t.shape[0], 4), 1)
          ).astype(jnp.float32)
    h = jnp.dot(oh, Wi_ref[...], preferred_element_type=jnp.float32) + bi_ref[...]
    f = ff_ref[...]
    diag = jnp.concatenate([f[:, 0:1], f[:, 4:5], f[:, 8:9]], axis=-1)
    frob = jnp.sqrt(jnp.sum(f * f, axis=-1, keepdims=True))
    trace = jnp.sum(diag, axis=-1, keepdims=True)
    det = (f[:, 0] * (f[:, 4] * f[:, 8] - f[:, 5] * f[:, 7])
           - f[:, 1] * (f[:, 3] * f[:, 8] - f[:, 5] * f[:, 6])
           + f[:, 2] * (f[:, 3] * f[:, 7] - f[:, 4] * f[:, 6]))[:, None]
    feats = jnp.concatenate([diag, frob, trace, det], axis=-1)
    o_ref[...] = h + jnp.dot(feats, Wf_ref[...],
                             preferred_element_type=jnp.float32) + bf_ref[...]


def _embed(types_pad, frames_pad, W_in, b_in, W_fr, b_fr):
    grid = (NP // NBK,)
    return pl.pallas_call(
        _embed_body,
        grid=grid,
        in_specs=[
            pl.BlockSpec((NBK, 1), lambda i: (i, 0)),
            pl.BlockSpec((NBK, 9), lambda i: (i, 0)),
            pl.BlockSpec((4, HID), lambda i: (0, 0)),
            pl.BlockSpec((1, HID), lambda i: (0, 0)),
            pl.BlockSpec((6, HID), lambda i: (0, 0)),
            pl.BlockSpec((1, HID), lambda i: (0, 0)),
        ],
        out_specs=pl.BlockSpec((NBK, HID), lambda i: (i, 0)),
        out_shape=jax.ShapeDtypeStruct((NP, HID), jnp.float32),
    )(types_pad, frames_pad, W_in, b_in[None, :], W_fr, b_fr[None, :])


def _ab_body(h_ref, Ws_ref, Wd_ref, b1_ref, a_ref, b_ref):
    h = h_ref[...]
    a_ref[...] = jnp.dot(h, Ws_ref[...], preferred_element_type=jnp.float32)
    b_ref[...] = jnp.dot(h, Wd_ref[...],
                         preferred_element_type=jnp.float32) + b1_ref[...]


def _ab(h, We1, be1):
    grid = (NP // NBK,)
    return pl.pallas_call(
        _ab_body,
        grid=grid,
        in_specs=[
            pl.BlockSpec((NBK, HID), lambda i: (i, 0)),
            pl.BlockSpec((HID, HID), lambda i: (0, 0)),
            pl.BlockSpec((HID, HID), lambda i: (0, 0)),
            pl.BlockSpec((1, HID), lambda i: (0, 0)),
        ],
        out_specs=[
            pl.BlockSpec((NBK, HID), lambda i: (i, 0)),
            pl.BlockSpec((NBK, HID), lambda i: (i, 0)),
        ],
        out_shape=[
            jax.ShapeDtypeStruct((NP, HID), jnp.float32),
            jax.ShapeDtypeStruct((NP, HID), jnp.float32),
        ],
    )(h, We1[:HID], We1[HID:2 * HID], be1[None, :])


def _edge_mlp_body(ag_ref, bg_ref, ef_ref, W1e_ref, W2_ref, b2_ref, o_ref):
    t = ag_ref[...] + bg_ref[...] + jnp.dot(
        ef_ref[...], W1e_ref[...], preferred_element_type=jnp.float32)
    t = jax.nn.silu(t)
    m = jnp.dot(t, W2_ref[...], preferred_element_type=jnp.float32) + b2_ref[...]
    o_ref[...] = jax.nn.silu(m)


def _edge_mlp(Ag, Bg, efeat, W1e, We2, be2):
    grid = (ME // EBK,)
    return pl.pallas_call(
        _edge_mlp_body,
        grid=grid,
        in_specs=[
            pl.BlockSpec((EBK, HID), lambda i: (i, 0)),
            pl.BlockSpec((EBK, HID), lambda i: (i, 0)),
            pl.BlockSpec((EBK, 1 + NRBF), lambda i: (i, 0)),
            pl.BlockSpec((1 + NRBF, HID), lambda i: (0, 0)),
            pl.BlockSpec((HID, HID), lambda i: (0, 0)),
            pl.BlockSpec((1, HID), lambda i: (0, 0)),
        ],
        out_specs=pl.BlockSpec((EBK, HID), lambda i: (i, 0)),
        out_shape=jax.ShapeDtypeStruct((ME, HID), jnp.float32),
    )(Ag, Bg, efeat, W1e, We2, be2[None, :])


def _node_upd_body(h_ref, g_ref, W1h_ref, W1a_ref, b1_ref, W2_ref, b2_ref,
                   o_ref):
    h = h_ref[...]
    u = jnp.dot(h, W1h_ref[...], preferred_element_type=jnp.float32)
    u = u + jnp.dot(g_ref[...], W1a_ref[...],
                    preferred_element_type=jnp.float32) + b1_ref[...]
    u = jax.nn.silu(u)
    u = jnp.dot(u, W2_ref[...], preferred_element_type=jnp.float32) + b2_ref[...]
    o_ref[...] = h + u


def _node_upd(h, agg, Wn1, bn1, Wn2, bn2):
    grid = (NP // NBK,)
    return pl.pallas_call(
        _node_upd_body,
        grid=grid,
        in_specs=[
            pl.BlockSpec((NBK, HID), lambda i: (i, 0)),
            pl.BlockSpec((NBK, HID), lambda i: (i, 0)),
            pl.BlockSpec((HID, HID), lambda i: (0, 0)),
            pl.BlockSpec((HID, HID), lambda i: (0, 0)),
            pl.BlockSpec((1, HID), lambda i: (0, 0)),
            pl.BlockSpec((HID, HID), lambda i: (0, 0)),
            pl.BlockSpec((1, HID), lambda i: (0, 0)),
        ],
        out_specs=pl.BlockSpec((NBK, HID), lambda i: (i, 0)),
        out_shape=jax.ShapeDtypeStruct((NP, HID), jnp.float32),
    )(h, agg, Wn1[:HID], Wn1[HID:], bn1[None, :], Wn2, bn2[None, :])


def _heads_body(h_ref, Wm_ref, bm_ref, Wl_ref, bl_ref, mu_ref, lv_ref):
    h = h_ref[...]
    mu_ref[...] = jnp.dot(h, Wm_ref[...],
                          preferred_element_type=jnp.float32) + bm_ref[...]
    lv = jnp.dot(h, Wl_ref[...], preferred_element_type=jnp.float32) + bl_ref[...]
    lv_ref[...] = jnp.clip(lv, -10.0, 2.0)


def _heads(h, W_mu, b_mu, W_lv, b_lv):
    grid = (NP // NBK,)
    return pl.pallas_call(
        _heads_body,
        grid=grid,
        in_specs=[
            pl.BlockSpec((NBK, HID), lambda i: (i, 0)),
            pl.BlockSpec((HID, LAT), lambda i: (0, 0)),
            pl.BlockSpec((1, LAT), lambda i: (0, 0)),
            pl.BlockSpec((HID, LAT), lambda i: (0, 0)),
            pl.BlockSpec((1, LAT), lambda i: (0, 0)),
        ],
        out_specs=[
            pl.BlockSpec((NBK, LAT), lambda i: (i, 0)),
            pl.BlockSpec((NBK, LAT), lambda i: (i, 0)),
        ],
        out_shape=[
            jax.ShapeDtypeStruct((NP, LAT), jnp.float32),
            jax.ShapeDtypeStruct((NP, LAT), jnp.float32),
        ],
    )(h, W_mu, b_mu[None, :], W_lv, b_lv[None, :])


def _edge_feats_body(d2_ref, o_ref):
    d2 = d2_ref[...]
    d = jnp.sqrt(d2)
    centers = jax.lax.broadcasted_iota(jnp.int32, (1, NRBF), 1).astype(
        jnp.float32) * (CUT / (NRBF - 1))
    width = CUT / NRBF
    rbf = jnp.exp(-((d - centers) ** 2) / (2.0 * width * width))
    o_ref[...] = jnp.concatenate([d2, rbf], axis=-1)


def _edge_feats(d2e):
    grid = (ME // EBK,)
    return pl.pallas_call(
        _edge_feats_body,
        grid=grid,
        in_specs=[pl.BlockSpec((EBK, 1), lambda i: (i, 0))],
        out_specs=pl.BlockSpec((EBK, 1 + NRBF), lambda i: (i, 0)),
        out_shape=jax.ShapeDtypeStruct((ME, 1 + NRBF), jnp.float32),
    )(d2e)


def kernel(x, node_types, frames, W_in, b_in, W_fr, b_fr, We1, be1, We2, be2,
           Wn1, bn1, Wn2, bn2, W_mu, b_mu, W_lv, b_lv):
    xc = x - jnp.mean(x, axis=0, keepdims=True)

    sq = jnp.sum(xc * xc, axis=1)
    d2m = sq[:, None] + sq[None, :] - 2.0 * (xc @ xc.T)
    mask = (d2m <= CUT * CUT) & (d2m > 1e-6)
    r0, r1 = jnp.nonzero(mask, size=ME, fill_value=NN)
    e_dst = r0
    e_src = r1

    xs = xc[jnp.minimum(e_src, NN - 1)]
    xd = xc[jnp.minimum(e_dst, NN - 1)]
    rel = xs - xd
    d2e = jnp.sum(rel * rel, axis=-1, keepdims=True)
    efeat = _edge_feats(d2e)

    types_pad = jnp.zeros((NP, 1), jnp.int32).at[:NN, 0].set(
        node_types.astype(jnp.int32))
    frames_pad = jnp.zeros((NP, 9), jnp.float32).at[:NN].set(
        frames.reshape(NN, 9))
    h = _embed(types_pad, frames_pad, W_in, b_in, W_fr, b_fr)

    for l in range(NL):
        A, B = _ab(h, We1[l], be1[l])
        Ag = A[jnp.minimum(e_src, NP - 1)]
        Bg = B[jnp.minimum(e_dst, NP - 1)]
        m = _edge_mlp(Ag, Bg, efeat, We1[l][2 * HID:], We2[l], be2[l])
        agg = jnp.zeros((NP, HID), jnp.float32).at[e_dst].add(m)
        h = _node_upd(h, agg, Wn1[l], bn1[l], Wn2[l], bn2[l])

    mu, lv = _heads(h, W_mu, b_mu, W_lv, b_lv)
    return (mu[:NN], lv[:NN])
